# Initial kernel scaffold; baseline (speedup 1.0000x reference)
import jax
import jax.numpy as jnp
from jax.experimental import pallas as pl


def kernel(x, nets, rho_params):
    raise NotImplementedError("write your pallas kernel here")


# speedup vs baseline: 1.6859x; 1.6859x over previous
import functools

import jax
import jax.numpy as jnp
from jax.experimental import pallas as pl


def _fused_body(x_ref, w1_ref, b1_ref, w2_ref, b2_ref, w3_ref, b3_ref,
                w4c_ref, bc_ref, wr2_ref, br2_ref, wr3_ref, br3_ref,
                wr4_ref, br4_ref, out_ref, *, ndim):
    x = x_ref[...]
    acc = None
    for d in range(ndim):
        col = x[:, d:d + 1]
        h = jnp.maximum(col * w1_ref[d:d + 1, :] + b1_ref[d:d + 1, :], 0.0)
        h = jnp.maximum(
            jnp.dot(h, w2_ref[d], preferred_element_type=jnp.float32)
            + b2_ref[d:d + 1, :], 0.0)
        h = jnp.maximum(
            jnp.dot(h, w3_ref[d], preferred_element_type=jnp.float32)
            + b3_ref[d:d + 1, :], 0.0)
        g = jnp.dot(h, w4c_ref[d], preferred_element_type=jnp.float32)
        acc = g if acc is None else acc + g
    h = jnp.maximum(acc + bc_ref[...], 0.0)
    h = jnp.maximum(
        jnp.dot(h, wr2_ref[...], preferred_element_type=jnp.float32)
        + br2_ref[...], 0.0)
    h = jnp.maximum(
        jnp.dot(h, wr3_ref[...], preferred_element_type=jnp.float32)
        + br3_ref[...], 0.0)
    out_ref[...] = (
        jnp.dot(h, wr4_ref[...], preferred_element_type=jnp.float32)
        + br4_ref[...])


def kernel(x, nets, rho_params):
    n, ndim = x.shape
    inter = nets[0][-1][0].shape[1]
    zdim = rho_params[-1][0].shape[1]

    w1 = jnp.concatenate([net[0][0] for net in nets], axis=0)
    b1 = jnp.stack([net[0][1] for net in nets], axis=0)
    w2 = jnp.stack([net[1][0] for net in nets], axis=0)
    b2 = jnp.stack([net[1][1] for net in nets], axis=0)
    w3 = jnp.stack([net[2][0] for net in nets], axis=0)
    b3 = jnp.stack([net[2][1] for net in nets], axis=0)
    w4 = jnp.stack([net[3][0] for net in nets], axis=0)
    b4 = jnp.stack([net[3][1] for net in nets], axis=0)

    wr1, br1 = rho_params[0]
    wr2, br2 = rho_params[1]
    wr3, br3 = rho_params[2]
    wr4, br4 = rho_params[3]

    w4c = jnp.einsum('dij,jk->dik', w4, wr1)
    bc = (jnp.sum(b4, axis=0) @ wr1 + br1)[None, :]

    blk = 1024
    n_pad = ((n + blk - 1) // blk) * blk
    xp = x if n_pad == n else jnp.pad(x, ((0, n_pad - n), (0, 0)))

    full = lambda a: pl.BlockSpec(a.shape, lambda i: (0,) * a.ndim)
    args = (w1, b1, w2, b2, w3, b3, w4c, bc,
            wr2, br2[None, :], wr3, br3[None, :], wr4, br4[None, :])

    out = pl.pallas_call(
        functools.partial(_fused_body, ndim=ndim),
        grid=(n_pad // blk,),
        in_specs=[pl.BlockSpec((blk, ndim), lambda i: (i, 0))]
                 + [full(a) for a in args],
        out_specs=pl.BlockSpec((blk, zdim), lambda i: (i, 0)),
        out_shape=jax.ShapeDtypeStruct((n_pad, zdim), jnp.float32),
    )(xp, *args)
    return out[:n] if n_pad != n else out
